# baseline (speedup 1.0000x reference)
import jax
import jax.numpy as jnp
from jax import lax
from jax.experimental import pallas as pl
from jax.experimental.pallas import tpu as pltpu
from jax.experimental.pallas import tpu_sc as plsc

_LATENT = 64
_BATCH = 16384
_LANES = 16

_info = plsc.get_sparse_core_info()
_NC, _NS = _info.num_cores, _info.num_subcores
_NW = _NC * _NS
_B_PER_W = _BATCH // _NW
_GROUPS = _B_PER_W // _LANES


def _mf_body(u_hbm, i_hbm, u_emb_hbm, i_emb_hbm, out_hbm,
             uidx_v, iidx_v, urows_v, irows_v, tr_v, out_v, sem_u, sem_i):
    wid = lax.axis_index("s") * _NC + lax.axis_index("c")
    base = wid * _B_PER_W

    pltpu.sync_copy(u_hbm.at[pl.ds(base, _B_PER_W)], uidx_v)
    pltpu.sync_copy(i_hbm.at[pl.ds(base, _B_PER_W)], iidx_v)
    cu = pltpu.async_copy(u_emb_hbm.at[uidx_v], urows_v, sem_u)
    ci = pltpu.async_copy(i_emb_hbm.at[iidx_v], irows_v, sem_i)
    cu.wait()
    ci.wait()

    col_ids = lax.iota(jnp.int32, _LANES) * _LANES

    def group(g, carry):
        rbase = g * _LANES
        for r in range(_LANES):
            row = rbase + r
            p = jnp.zeros((_LANES,), jnp.float32)
            for c in range(_LATENT // _LANES):
                ue = urows_v[row, pl.ds(c * _LANES, _LANES)]
                ie = irows_v[row, pl.ds(c * _LANES, _LANES)]
                p = p + ue * ie
            plsc.store_scatter(tr_v, [col_ids + r], p)
        acc = tr_v[pl.ds(0, _LANES)]
        for l in range(1, _LANES):
            acc = acc + tr_v[pl.ds(l * _LANES, _LANES)]
        out_v[pl.ds(rbase, _LANES)] = acc
        return carry

    lax.fori_loop(0, _GROUPS, group, 0)
    pltpu.sync_copy(out_v, out_hbm.at[pl.ds(base, _B_PER_W)])


@jax.jit
def _mf(u, i, u_emb, i_emb):
    mesh = plsc.VectorSubcoreMesh(core_axis_name="c", subcore_axis_name="s")
    return pl.kernel(
        _mf_body,
        mesh=mesh,
        compiler_params=pltpu.CompilerParams(
            needs_layout_passes=False, use_tc_tiling_on_sc=False),
        out_type=jax.ShapeDtypeStruct((_BATCH,), jnp.float32),
        scratch_types=[
            pltpu.VMEM((_B_PER_W,), jnp.int32),
            pltpu.VMEM((_B_PER_W,), jnp.int32),
            pltpu.VMEM((_B_PER_W, _LATENT), jnp.float32),
            pltpu.VMEM((_B_PER_W, _LATENT), jnp.float32),
            pltpu.VMEM((_LANES * _LANES,), jnp.float32),
            pltpu.VMEM((_B_PER_W,), jnp.float32),
            pltpu.SemaphoreType.DMA,
            pltpu.SemaphoreType.DMA,
        ],
    )(u, i, u_emb, i_emb)


def kernel(u, i, u_emb, i_emb):
    return _mf(u, i, u_emb, i_emb)
